# baseline (speedup 1.0000x reference)
import functools

import jax
import jax.numpy as jnp
from jax import lax
from jax.experimental import pallas as pl
from jax.experimental.pallas import tpu as pltpu
from jax.experimental.pallas import tpu_sc as plsc

T = 2048
D = 1024
E = 16
K = 2
FF = 512
TK = T * K

BT = 128
NB = 48
NP = NB * BT

BTR = 256
NC, NS, L = 2, 16, 16
NW = NC * NS


def _router_body(x_ref, wr_ref, bias_ref, scores_ref, inds_ref, rank_ref,
                 cnt_ref, carry_s):
    t = pl.program_id(0)

    @pl.when(t == 0)
    def _init():
        carry_s[...] = jnp.zeros_like(carry_s)

    xb = x_ref[...]
    logits = lax.dot_general(xb, wr_ref[...], (((1,), (1,)), ((), ())),
                             preferred_element_type=jnp.float32)
    m = jnp.max(logits, axis=-1, keepdims=True)
    p = jnp.exp(logits - m)
    gates = p / jnp.sum(p, axis=-1, keepdims=True)
    g = gates + bias_ref[...]
    iota = lax.broadcasted_iota(jnp.int32, (BTR, E), 1)
    m1 = jnp.max(g, axis=-1, keepdims=True)
    i1 = jnp.min(jnp.where(g == m1, iota, E), axis=-1, keepdims=True)
    g2 = jnp.where(iota == i1, -1e30, g)
    m2 = jnp.max(g2, axis=-1, keepdims=True)
    i2 = jnp.min(jnp.where(g2 == m2, iota, E), axis=-1, keepdims=True)
    denom = m1 + m2 + 1e-20
    s1 = m1 / denom
    s2 = m2 / denom

    a1 = (iota == i1).astype(jnp.float32)
    a2 = (iota == i2).astype(jnp.float32)
    both = a1 + a2
    ri = lax.broadcasted_iota(jnp.int32, (BTR, BTR), 0)
    ci = lax.broadcasted_iota(jnp.int32, (BTR, BTR), 1)
    tril = (ri > ci).astype(jnp.float32)
    cum = lax.dot_general(tril, both, (((1,), (0,)), ((), ())),
                          preferred_element_type=jnp.float32)
    carry = carry_s[...]
    r0 = jnp.sum((cum + carry) * a1, axis=1, keepdims=True)
    r1 = jnp.sum((cum + carry) * a2, axis=1, keepdims=True)
    carry_new = carry + jnp.sum(both, axis=0, keepdims=True)
    carry_s[...] = carry_new

    scores_ref[...] = jnp.concatenate([s1, s2], axis=1)
    inds_ref[...] = jnp.concatenate([i1, i2], axis=1)
    rank_ref[...] = jnp.concatenate([r0, r1], axis=1).astype(jnp.int32)
    cnt_ref[...] = carry_new.astype(jnp.int32)


def _router(x, Wr, bias2):
    return pl.pallas_call(
        _router_body,
        grid=(T // BTR,),
        in_specs=[
            pl.BlockSpec((BTR, D), lambda t: (t, 0)),
            pl.BlockSpec((E, D), lambda t: (0, 0)),
            pl.BlockSpec((1, E), lambda t: (0, 0)),
        ],
        out_specs=[
            pl.BlockSpec((BTR, K), lambda t: (t, 0)),
            pl.BlockSpec((BTR, K), lambda t: (t, 0)),
            pl.BlockSpec((BTR, K), lambda t: (t, 0)),
            pl.BlockSpec((1, E), lambda t: (0, 0)),
        ],
        out_shape=[
            jax.ShapeDtypeStruct((T, K), jnp.float32),
            jax.ShapeDtypeStruct((T, K), jnp.int32),
            jax.ShapeDtypeStruct((T, K), jnp.int32),
            jax.ShapeDtypeStruct((1, E), jnp.int32),
        ],
        scratch_shapes=[pltpu.VMEM((1, E), jnp.float32)],
    )(x, Wr, bias2)


def _finalize_body(cnt_ref, inds_ref, rank_ref, pos_ref, blk_ref, nb_ref):
    cnt = cnt_ref[...]
    nbe = (cnt + (BT - 1)) // BT
    nbef = nbe.astype(jnp.float32)
    er = lax.broadcasted_iota(jnp.int32, (E, E), 0)
    ec = lax.broadcasted_iota(jnp.int32, (E, E), 1)
    triu = (er < ec).astype(jnp.float32)
    pblk = lax.dot_general(nbef, triu, (((1,), (0,)), ((), ())),
                           preferred_element_type=jnp.float32)
    pstart = pblk * BT

    inds = inds_ref[...]
    rank = rank_ref[...]
    iota_e = lax.broadcasted_iota(jnp.int32, (T, E), 1)
    iota_k = lax.broadcasted_iota(jnp.int32, (T, K), 1)
    i1 = jnp.sum(jnp.where(iota_k == 0, inds, 0), axis=1, keepdims=True)
    i2 = jnp.sum(jnp.where(iota_k == 1, inds, 0), axis=1, keepdims=True)
    r1 = jnp.sum(jnp.where(iota_k == 0, rank, 0), axis=1, keepdims=True)
    r2 = jnp.sum(jnp.where(iota_k == 1, rank, 0), axis=1, keepdims=True)
    pg1 = jnp.sum(jnp.where(i1 == iota_e, pstart, 0.0), axis=1, keepdims=True)
    pg2 = jnp.sum(jnp.where(i2 == iota_e, pstart, 0.0), axis=1, keepdims=True)
    pos0 = pg1.astype(jnp.int32) + r1
    pos1 = pg2.astype(jnp.int32) + r2
    pos_ref[...] = jnp.concatenate([pos0, pos1], axis=1)

    cb = pblk + nbef
    bf = lax.broadcasted_iota(jnp.int32, (NB, E), 0).astype(jnp.float32)
    blk = jnp.sum((cb <= bf).astype(jnp.int32), axis=1, keepdims=True)
    blk_ref[...] = jnp.minimum(blk, E - 1)
    nb_ref[...] = jnp.sum(nbe, axis=1, keepdims=True)


def _finalize(cnt, inds, rank):
    return pl.pallas_call(
        _finalize_body,
        in_specs=[
            pl.BlockSpec((1, E), lambda: (0, 0)),
            pl.BlockSpec((T, K), lambda: (0, 0)),
            pl.BlockSpec((T, K), lambda: (0, 0)),
        ],
        out_specs=[
            pl.BlockSpec((T, K), lambda: (0, 0)),
            pl.BlockSpec((NB, 1), lambda: (0, 0)),
            pl.BlockSpec((1, 1), lambda: (0, 0)),
        ],
        out_shape=[
            jax.ShapeDtypeStruct((T, K), jnp.int32),
            jax.ShapeDtypeStruct((NB, 1), jnp.int32),
            jax.ShapeDtypeStruct((1, 1), jnp.int32),
        ],
    )(cnt, inds, rank)


@functools.cache
def _make_sc_tok_build():
    mesh = plsc.VectorSubcoreMesh(core_axis_name="c", subcore_axis_name="s")

    @functools.partial(
        pl.kernel,
        mesh=mesh,
        out_type=jax.ShapeDtypeStruct((NP,), jnp.int32),
        scratch_types=[
            pltpu.VMEM((TK,), jnp.int32),
            pltpu.VMEM((NP,), jnp.int32),
        ],
        compiler_params=pltpu.CompilerParams(needs_layout_passes=False),
    )
    def _sc_tok_build(pos_hbm, tok_hbm, pos_v, tok_v):
        wid = lax.axis_index("s") * NC + lax.axis_index("c")

        @pl.when(wid == 0)
        def _():
            zero = jnp.zeros((L,), jnp.int32)
            for c in range(NP // L):
                tok_v[pl.ds(c * L, L)] = zero
            pltpu.sync_copy(pos_hbm, pos_v)
            iota = lax.broadcasted_iota(jnp.int32, (L,), 0)
            for c in range(TK // L):
                pv = pos_v[pl.ds(c * L, L)]
                tv = lax.shift_right_logical(iota + (c * L), 1)
                plsc.store_scatter(tok_v, [pv], tv)
            pltpu.sync_copy(tok_v, tok_hbm)

    return _sc_tok_build


@functools.cache
def _make_sc_row_gather(nrows, ncols, chunk):
    per_w = nrows // NW
    nch = per_w // chunk
    mesh = plsc.VectorSubcoreMesh(core_axis_name="c", subcore_axis_name="s")

    @functools.partial(
        pl.kernel,
        mesh=mesh,
        out_type=jax.ShapeDtypeStruct((nrows, ncols), jnp.float32),
        scratch_types=[
            pltpu.VMEM((chunk,), jnp.int32),
            pltpu.VMEM((chunk, ncols), jnp.float32),
            pltpu.SemaphoreType.DMA,
        ],
        compiler_params=pltpu.CompilerParams(needs_layout_passes=False),
    )
    def _gather(idx_hbm, table_hbm, out_hbm, idx_v, rows_v, sem):
        wid = lax.axis_index("s") * NC + lax.axis_index("c")
        base = wid * per_w
        for c in range(nch):
            off = base + c * chunk
            pltpu.sync_copy(idx_hbm.at[pl.ds(off, chunk)], idx_v)
            pltpu.async_copy(table_hbm.at[idx_v], rows_v, sem).wait()
            pltpu.sync_copy(rows_v, out_hbm.at[pl.ds(off, chunk)])

    return _gather


def _ffn_body(blk_ref, nb_ref, xs_ref, wg_ref, wu_ref, wd_ref, yp_ref):
    b = pl.program_id(0)

    @pl.when(b < nb_ref[0])
    def _():
        xb = xs_ref[...]
        hg = lax.dot_general(xb, wg_ref[0], (((1,), (1,)), ((), ())),
                             preferred_element_type=jnp.float32)
        hu = lax.dot_general(xb, wu_ref[0], (((1,), (1,)), ((), ())),
                             preferred_element_type=jnp.float32)
        h = hg * lax.logistic(hg) * hu
        yp_ref[...] = lax.dot_general(h, wd_ref[0], (((1,), (1,)), ((), ())),
                                      preferred_element_type=jnp.float32)


def _grouped_ffn(blk, nb, xs, Wg, Wu, Wd):
    grid_spec = pltpu.PrefetchScalarGridSpec(
        num_scalar_prefetch=2,
        grid=(NB,),
        in_specs=[
            pl.BlockSpec((BT, D), lambda b, blk, nb: (b, 0)),
            pl.BlockSpec((1, FF, D), lambda b, blk, nb: (blk[b], 0, 0)),
            pl.BlockSpec((1, FF, D), lambda b, blk, nb: (blk[b], 0, 0)),
            pl.BlockSpec((1, D, FF), lambda b, blk, nb: (blk[b], 0, 0)),
        ],
        out_specs=pl.BlockSpec((BT, D), lambda b, blk, nb: (b, 0)),
    )
    return pl.pallas_call(
        _ffn_body,
        grid_spec=grid_spec,
        out_shape=jax.ShapeDtypeStruct((NP, D), jnp.float32),
    )(blk, nb, xs, Wg, Wu, Wd)


def _combine_body(ypt_ref, scores_ref, y_ref):
    s = jnp.expand_dims(scores_ref[...], -1)
    y_ref[...] = jnp.sum(ypt_ref[...] * s, axis=1)


def _combine(ypt, scores):
    return pl.pallas_call(
        _combine_body,
        grid=(T // BTR,),
        in_specs=[
            pl.BlockSpec((BTR, K, D), lambda t: (t, 0, 0)),
            pl.BlockSpec((BTR, K), lambda t: (t, 0)),
        ],
        out_specs=pl.BlockSpec((BTR, D), lambda t: (t, 0)),
        out_shape=jax.ShapeDtypeStruct((T, D), jnp.float32),
    )(ypt, scores)


def kernel(x, Wr, Wg, Wu, Wd, expert_bias):
    bias2 = expert_bias.reshape(1, E)
    scores, inds, rank, cnt = _router(x, Wr, bias2)
    pos, blk, nb = _finalize(cnt, inds, rank)
    posflat = pos.reshape(TK)
    tok = _make_sc_tok_build()(posflat)
    xs = _make_sc_row_gather(NP, D, 32)(tok, x)
    yp = _grouped_ffn(blk.reshape(NB), nb.reshape(1), xs, Wg, Wu, Wd)
    ypt = _make_sc_row_gather(TK, D, 32)(posflat, yp)
    y = _combine(ypt.reshape(T, K, D), scores)
    return y


# speedup vs baseline: 1.0033x; 1.0033x over previous
import functools

import jax
import jax.numpy as jnp
from jax import lax
from jax.experimental import pallas as pl
from jax.experimental.pallas import tpu as pltpu
from jax.experimental.pallas import tpu_sc as plsc

T = 2048
D = 1024
E = 16
K = 2
FF = 512
TK = T * K

BT = 128
NB = 48
NP = NB * BT

BTR = 256
NC, NS, L = 2, 16, 16
NW = NC * NS


def _router_body(x_ref, wr_ref, bias_ref, scores_ref, inds_ref, rank_ref,
                 cnt_ref, carry_s):
    t = pl.program_id(0)

    @pl.when(t == 0)
    def _init():
        carry_s[...] = jnp.zeros_like(carry_s)

    xb = x_ref[...]
    logits = lax.dot_general(xb, wr_ref[...], (((1,), (1,)), ((), ())),
                             preferred_element_type=jnp.float32)
    m = jnp.max(logits, axis=-1, keepdims=True)
    p = jnp.exp(logits - m)
    gates = p / jnp.sum(p, axis=-1, keepdims=True)
    g = gates + bias_ref[...]
    iota = lax.broadcasted_iota(jnp.int32, (BTR, E), 1)
    m1 = jnp.max(g, axis=-1, keepdims=True)
    i1 = jnp.min(jnp.where(g == m1, iota, E), axis=-1, keepdims=True)
    g2 = jnp.where(iota == i1, -1e30, g)
    m2 = jnp.max(g2, axis=-1, keepdims=True)
    i2 = jnp.min(jnp.where(g2 == m2, iota, E), axis=-1, keepdims=True)
    denom = m1 + m2 + 1e-20
    s1 = m1 / denom
    s2 = m2 / denom

    a1 = (iota == i1).astype(jnp.float32)
    a2 = (iota == i2).astype(jnp.float32)
    both = a1 + a2
    ri = lax.broadcasted_iota(jnp.int32, (BTR, BTR), 0)
    ci = lax.broadcasted_iota(jnp.int32, (BTR, BTR), 1)
    tril = (ri > ci).astype(jnp.float32)
    cum = lax.dot_general(tril, both, (((1,), (0,)), ((), ())),
                          preferred_element_type=jnp.float32)
    carry = carry_s[...]
    r0 = jnp.sum((cum + carry) * a1, axis=1, keepdims=True)
    r1 = jnp.sum((cum + carry) * a2, axis=1, keepdims=True)
    carry_new = carry + jnp.sum(both, axis=0, keepdims=True)
    carry_s[...] = carry_new

    scores_ref[...] = jnp.concatenate([s1, s2], axis=1)
    inds_ref[...] = jnp.concatenate([i1, i2], axis=1)
    rank_ref[...] = jnp.concatenate([r0, r1], axis=1).astype(jnp.int32)
    cnt_ref[...] = carry_new.astype(jnp.int32)


def _router(x, Wr, bias2):
    return pl.pallas_call(
        _router_body,
        grid=(T // BTR,),
        in_specs=[
            pl.BlockSpec((BTR, D), lambda t: (t, 0)),
            pl.BlockSpec((E, D), lambda t: (0, 0)),
            pl.BlockSpec((1, E), lambda t: (0, 0)),
        ],
        out_specs=[
            pl.BlockSpec((BTR, K), lambda t: (t, 0)),
            pl.BlockSpec((BTR, K), lambda t: (t, 0)),
            pl.BlockSpec((BTR, K), lambda t: (t, 0)),
            pl.BlockSpec((1, E), lambda t: (0, 0)),
        ],
        out_shape=[
            jax.ShapeDtypeStruct((T, K), jnp.float32),
            jax.ShapeDtypeStruct((T, K), jnp.int32),
            jax.ShapeDtypeStruct((T, K), jnp.int32),
            jax.ShapeDtypeStruct((1, E), jnp.int32),
        ],
        scratch_shapes=[pltpu.VMEM((1, E), jnp.float32)],
    )(x, Wr, bias2)


def _finalize_body(cnt_ref, inds_ref, rank_ref, pos_ref, blk_ref, nb_ref):
    cnt = cnt_ref[...]
    nbe = (cnt + (BT - 1)) // BT
    nbef = nbe.astype(jnp.float32)
    er = lax.broadcasted_iota(jnp.int32, (E, E), 0)
    ec = lax.broadcasted_iota(jnp.int32, (E, E), 1)
    triu = (er < ec).astype(jnp.float32)
    pblk = lax.dot_general(nbef, triu, (((1,), (0,)), ((), ())),
                           preferred_element_type=jnp.float32)
    pstart = pblk * BT

    inds = inds_ref[...]
    rank = rank_ref[...]
    iota_e = lax.broadcasted_iota(jnp.int32, (T, E), 1)
    iota_k = lax.broadcasted_iota(jnp.int32, (T, K), 1)
    i1 = jnp.sum(jnp.where(iota_k == 0, inds, 0), axis=1, keepdims=True)
    i2 = jnp.sum(jnp.where(iota_k == 1, inds, 0), axis=1, keepdims=True)
    r1 = jnp.sum(jnp.where(iota_k == 0, rank, 0), axis=1, keepdims=True)
    r2 = jnp.sum(jnp.where(iota_k == 1, rank, 0), axis=1, keepdims=True)
    pg1 = jnp.sum(jnp.where(i1 == iota_e, pstart, 0.0), axis=1, keepdims=True)
    pg2 = jnp.sum(jnp.where(i2 == iota_e, pstart, 0.0), axis=1, keepdims=True)
    pos0 = pg1.astype(jnp.int32) + r1
    pos1 = pg2.astype(jnp.int32) + r2
    pos_ref[...] = jnp.concatenate([pos0, pos1], axis=1)

    cb = pblk + nbef
    bf = lax.broadcasted_iota(jnp.int32, (NB, E), 0).astype(jnp.float32)
    blk = jnp.sum((cb <= bf).astype(jnp.int32), axis=1, keepdims=True)
    blk_ref[...] = jnp.minimum(blk, E - 1)
    nb_ref[...] = jnp.sum(nbe, axis=1, keepdims=True)


def _finalize(cnt, inds, rank):
    return pl.pallas_call(
        _finalize_body,
        in_specs=[
            pl.BlockSpec((1, E), lambda: (0, 0)),
            pl.BlockSpec((T, K), lambda: (0, 0)),
            pl.BlockSpec((T, K), lambda: (0, 0)),
        ],
        out_specs=[
            pl.BlockSpec((T, K), lambda: (0, 0)),
            pl.BlockSpec((NB, 1), lambda: (0, 0)),
            pl.BlockSpec((1, 1), lambda: (0, 0)),
        ],
        out_shape=[
            jax.ShapeDtypeStruct((T, K), jnp.int32),
            jax.ShapeDtypeStruct((NB, 1), jnp.int32),
            jax.ShapeDtypeStruct((1, 1), jnp.int32),
        ],
    )(cnt, inds, rank)


@functools.cache
def _make_sc_tok_build():
    mesh = plsc.VectorSubcoreMesh(core_axis_name="c", subcore_axis_name="s")

    @functools.partial(
        pl.kernel,
        mesh=mesh,
        out_type=jax.ShapeDtypeStruct((NP,), jnp.int32),
        scratch_types=[
            pltpu.VMEM((TK,), jnp.int32),
            pltpu.VMEM((NP,), jnp.int32),
        ],
        compiler_params=pltpu.CompilerParams(needs_layout_passes=False),
    )
    def _sc_tok_build(pos_hbm, tok_hbm, pos_v, tok_v):
        wid = lax.axis_index("s") * NC + lax.axis_index("c")

        @pl.when(wid == 0)
        def _():
            zero = jnp.zeros((L,), jnp.int32)
            for c in range(NP // L):
                tok_v[pl.ds(c * L, L)] = zero
            pltpu.sync_copy(pos_hbm, pos_v)
            iota = lax.broadcasted_iota(jnp.int32, (L,), 0)
            for c in range(TK // L):
                pv = pos_v[pl.ds(c * L, L)]
                tv = lax.shift_right_logical(iota + (c * L), 1)
                plsc.store_scatter(tok_v, [pv], tv)
            pltpu.sync_copy(tok_v, tok_hbm)

    return _sc_tok_build


@functools.cache
def _make_sc_row_gather(nrows, ncols, chunk):
    per_w = nrows // NW
    nch = per_w // chunk
    mesh = plsc.VectorSubcoreMesh(core_axis_name="c", subcore_axis_name="s")

    @functools.partial(
        pl.kernel,
        mesh=mesh,
        out_type=jax.ShapeDtypeStruct((nrows, ncols), jnp.float32),
        scratch_types=[
            pltpu.VMEM((per_w,), jnp.int32),
            pltpu.VMEM((chunk, ncols), jnp.float32),
            pltpu.VMEM((chunk, ncols), jnp.float32),
            pltpu.SemaphoreType.DMA,
            pltpu.SemaphoreType.DMA,
        ],
        compiler_params=pltpu.CompilerParams(needs_layout_passes=False),
    )
    def _gather(idx_hbm, table_hbm, out_hbm, idx_v, buf0, buf1, sem_g, sem_w):
        wid = lax.axis_index("s") * NC + lax.axis_index("c")
        base = wid * per_w
        pltpu.sync_copy(idx_hbm.at[pl.ds(base, per_w)], idx_v)
        bufs = [buf0, buf1]
        g = [None] * nch
        w = [None] * nch
        g[0] = pltpu.async_copy(
            table_hbm.at[idx_v.at[pl.ds(0, chunk)]], bufs[0], sem_g)
        for c in range(nch):
            g[c].wait()
            if c + 1 < nch:
                if c >= 1:
                    w[c - 1].wait()
                g[c + 1] = pltpu.async_copy(
                    table_hbm.at[idx_v.at[pl.ds((c + 1) * chunk, chunk)]],
                    bufs[(c + 1) % 2], sem_g)
            w[c] = pltpu.async_copy(
                bufs[c % 2], out_hbm.at[pl.ds(base + c * chunk, chunk)], sem_w)
        if nch >= 2:
            w[nch - 2].wait()
        w[nch - 1].wait()

    return _gather


def _ffn_body(blk_ref, nb_ref, xs_ref, wg_ref, wu_ref, wd_ref, yp_ref):
    b = pl.program_id(0)

    @pl.when(b < nb_ref[0])
    def _():
        xb = xs_ref[...]
        hg = lax.dot_general(xb, wg_ref[0], (((1,), (1,)), ((), ())),
                             preferred_element_type=jnp.float32)
        hu = lax.dot_general(xb, wu_ref[0], (((1,), (1,)), ((), ())),
                             preferred_element_type=jnp.float32)
        h = hg * lax.logistic(hg) * hu
        yp_ref[...] = lax.dot_general(h, wd_ref[0], (((1,), (1,)), ((), ())),
                                      preferred_element_type=jnp.float32)


def _grouped_ffn(blk, nb, xs, Wg, Wu, Wd):
    grid_spec = pltpu.PrefetchScalarGridSpec(
        num_scalar_prefetch=2,
        grid=(NB,),
        in_specs=[
            pl.BlockSpec((BT, D), lambda b, blk, nb: (b, 0)),
            pl.BlockSpec((1, FF, D), lambda b, blk, nb: (blk[b], 0, 0)),
            pl.BlockSpec((1, FF, D), lambda b, blk, nb: (blk[b], 0, 0)),
            pl.BlockSpec((1, D, FF), lambda b, blk, nb: (blk[b], 0, 0)),
        ],
        out_specs=pl.BlockSpec((BT, D), lambda b, blk, nb: (b, 0)),
    )
    return pl.pallas_call(
        _ffn_body,
        grid_spec=grid_spec,
        out_shape=jax.ShapeDtypeStruct((NP, D), jnp.float32),
    )(blk, nb, xs, Wg, Wu, Wd)


def _combine_body(ypt_ref, scores_ref, y_ref):
    s = jnp.expand_dims(scores_ref[...], -1)
    y_ref[...] = jnp.sum(ypt_ref[...] * s, axis=1)


def _combine(ypt, scores):
    return pl.pallas_call(
        _combine_body,
        grid=(T // BTR,),
        in_specs=[
            pl.BlockSpec((BTR, K, D), lambda t: (t, 0, 0)),
            pl.BlockSpec((BTR, K), lambda t: (t, 0)),
        ],
        out_specs=pl.BlockSpec((BTR, D), lambda t: (t, 0)),
        out_shape=jax.ShapeDtypeStruct((T, D), jnp.float32),
    )(ypt, scores)


def kernel(x, Wr, Wg, Wu, Wd, expert_bias):
    bias2 = expert_bias.reshape(1, E)
    scores, inds, rank, cnt = _router(x, Wr, bias2)
    pos, blk, nb = _finalize(cnt, inds, rank)
    posflat = pos.reshape(TK)
    tok = _make_sc_tok_build()(posflat)
    xs = _make_sc_row_gather(NP, D, 48)(tok, x)
    yp = _grouped_ffn(blk.reshape(NB), nb.reshape(1), xs, Wg, Wu, Wd)
    ypt = _make_sc_row_gather(TK, D, 32)(posflat, yp)
    y = _combine(ypt.reshape(T, K, D), scores)
    return y


# speedup vs baseline: 1.4677x; 1.4629x over previous
import functools

import jax
import jax.numpy as jnp
from jax import lax
from jax.experimental import pallas as pl
from jax.experimental.pallas import tpu as pltpu
from jax.experimental.pallas import tpu_sc as plsc

T = 2048
D = 1024
E = 16
K = 2
FF = 512
TK = T * K

BT = 128
NB = 48
NP = NB * BT

BTR = 256
NC, NS, L = 2, 16, 16
NW = NC * NS


def _router_body(x_ref, wr_ref, bias_ref, scores_ref, inds_ref, rank_ref,
                 cnt_ref, carry_s):
    t = pl.program_id(0)

    @pl.when(t == 0)
    def _init():
        carry_s[...] = jnp.zeros_like(carry_s)

    xb = x_ref[...]
    logits = lax.dot_general(xb, wr_ref[...], (((1,), (1,)), ((), ())),
                             preferred_element_type=jnp.float32)
    m = jnp.max(logits, axis=-1, keepdims=True)
    p = jnp.exp(logits - m)
    gates = p / jnp.sum(p, axis=-1, keepdims=True)
    g = gates + bias_ref[...]
    iota = lax.broadcasted_iota(jnp.int32, (BTR, E), 1)
    m1 = jnp.max(g, axis=-1, keepdims=True)
    i1 = jnp.min(jnp.where(g == m1, iota, E), axis=-1, keepdims=True)
    g2 = jnp.where(iota == i1, -1e30, g)
    m2 = jnp.max(g2, axis=-1, keepdims=True)
    i2 = jnp.min(jnp.where(g2 == m2, iota, E), axis=-1, keepdims=True)
    denom = m1 + m2 + 1e-20
    s1 = m1 / denom
    s2 = m2 / denom

    a1 = (iota == i1).astype(jnp.float32)
    a2 = (iota == i2).astype(jnp.float32)
    both = a1 + a2
    ri = lax.broadcasted_iota(jnp.int32, (BTR, BTR), 0)
    ci = lax.broadcasted_iota(jnp.int32, (BTR, BTR), 1)
    tril = (ri > ci).astype(jnp.float32)
    cum = lax.dot_general(tril, both, (((1,), (0,)), ((), ())),
                          preferred_element_type=jnp.float32)
    carry = carry_s[...]
    r0 = jnp.sum((cum + carry) * a1, axis=1, keepdims=True)
    r1 = jnp.sum((cum + carry) * a2, axis=1, keepdims=True)
    carry_new = carry + jnp.sum(both, axis=0, keepdims=True)
    carry_s[...] = carry_new

    scores_ref[...] = jnp.concatenate([s1, s2], axis=1)
    inds_ref[...] = jnp.concatenate([i1, i2], axis=1)
    rank_ref[...] = jnp.concatenate([r0, r1], axis=1).astype(jnp.int32)
    cnt_ref[...] = carry_new.astype(jnp.int32)


def _router(x, Wr, bias2):
    return pl.pallas_call(
        _router_body,
        grid=(T // BTR,),
        in_specs=[
            pl.BlockSpec((BTR, D), lambda t: (t, 0)),
            pl.BlockSpec((E, D), lambda t: (0, 0)),
            pl.BlockSpec((1, E), lambda t: (0, 0)),
        ],
        out_specs=[
            pl.BlockSpec((BTR, K), lambda t: (t, 0)),
            pl.BlockSpec((BTR, K), lambda t: (t, 0)),
            pl.BlockSpec((BTR, K), lambda t: (t, 0)),
            pl.BlockSpec((1, E), lambda t: (0, 0)),
        ],
        out_shape=[
            jax.ShapeDtypeStruct((T, K), jnp.float32),
            jax.ShapeDtypeStruct((T, K), jnp.int32),
            jax.ShapeDtypeStruct((T, K), jnp.int32),
            jax.ShapeDtypeStruct((1, E), jnp.int32),
        ],
        scratch_shapes=[pltpu.VMEM((1, E), jnp.float32)],
    )(x, Wr, bias2)


def _finalize_body(cnt_ref, inds_ref, rank_ref, pos_ref, blk_ref, nb_ref):
    cnt = cnt_ref[...]
    nbe = (cnt + (BT - 1)) // BT
    nbef = nbe.astype(jnp.float32)
    er = lax.broadcasted_iota(jnp.int32, (E, E), 0)
    ec = lax.broadcasted_iota(jnp.int32, (E, E), 1)
    triu = (er < ec).astype(jnp.float32)
    pblk = lax.dot_general(nbef, triu, (((1,), (0,)), ((), ())),
                           preferred_element_type=jnp.float32)
    pstart = pblk * BT

    inds = inds_ref[...]
    rank = rank_ref[...]
    iota_e = lax.broadcasted_iota(jnp.int32, (T, E), 1)
    iota_k = lax.broadcasted_iota(jnp.int32, (T, K), 1)
    i1 = jnp.sum(jnp.where(iota_k == 0, inds, 0), axis=1, keepdims=True)
    i2 = jnp.sum(jnp.where(iota_k == 1, inds, 0), axis=1, keepdims=True)
    r1 = jnp.sum(jnp.where(iota_k == 0, rank, 0), axis=1, keepdims=True)
    r2 = jnp.sum(jnp.where(iota_k == 1, rank, 0), axis=1, keepdims=True)
    pg1 = jnp.sum(jnp.where(i1 == iota_e, pstart, 0.0), axis=1, keepdims=True)
    pg2 = jnp.sum(jnp.where(i2 == iota_e, pstart, 0.0), axis=1, keepdims=True)
    pos0 = pg1.astype(jnp.int32) + r1
    pos1 = pg2.astype(jnp.int32) + r2
    pos_ref[...] = jnp.concatenate([pos0, pos1], axis=1)

    cb = pblk + nbef
    bf = lax.broadcasted_iota(jnp.int32, (NB, E), 0).astype(jnp.float32)
    blk = jnp.sum((cb <= bf).astype(jnp.int32), axis=1, keepdims=True)
    blk_ref[...] = jnp.minimum(blk, E - 1)
    nb_ref[...] = jnp.sum(nbe, axis=1, keepdims=True)


def _finalize(cnt, inds, rank):
    return pl.pallas_call(
        _finalize_body,
        in_specs=[
            pl.BlockSpec((1, E), lambda: (0, 0)),
            pl.BlockSpec((T, K), lambda: (0, 0)),
            pl.BlockSpec((T, K), lambda: (0, 0)),
        ],
        out_specs=[
            pl.BlockSpec((T, K), lambda: (0, 0)),
            pl.BlockSpec((NB, 1), lambda: (0, 0)),
            pl.BlockSpec((1, 1), lambda: (0, 0)),
        ],
        out_shape=[
            jax.ShapeDtypeStruct((T, K), jnp.int32),
            jax.ShapeDtypeStruct((NB, 1), jnp.int32),
            jax.ShapeDtypeStruct((1, 1), jnp.int32),
        ],
    )(cnt, inds, rank)


@functools.cache
def _make_sc_tok_build():
    mesh = plsc.VectorSubcoreMesh(core_axis_name="c", subcore_axis_name="s")

    @functools.partial(
        pl.kernel,
        mesh=mesh,
        out_type=jax.ShapeDtypeStruct((NP,), jnp.int32),
        scratch_types=[
            pltpu.VMEM((TK,), jnp.int32),
            pltpu.VMEM((NP,), jnp.int32),
        ],
        compiler_params=pltpu.CompilerParams(needs_layout_passes=False),
    )
    def _sc_tok_build(pos_hbm, tok_hbm, pos_v, tok_v):
        wid = lax.axis_index("s") * NC + lax.axis_index("c")

        @pl.when(wid == 0)
        def _():
            iota0 = lax.broadcasted_iota(jnp.int32, (L,), 0)
            for c in range(NP // L):
                tok_v[pl.ds(c * L, L)] = (iota0 + c * L) & (T - 1)
            pltpu.sync_copy(pos_hbm, pos_v)
            iota = lax.broadcasted_iota(jnp.int32, (L,), 0)
            for c in range(TK // L):
                pv = pos_v[pl.ds(c * L, L)]
                tv = lax.shift_right_logical(iota + (c * L), 1)
                plsc.store_scatter(tok_v, [pv], tv)
            pltpu.sync_copy(tok_v, tok_hbm)

    return _sc_tok_build


@functools.cache
def _make_sc_row_gather(nrows, ncols, chunk):
    per_w = nrows // NW
    nch = per_w // chunk
    mesh = plsc.VectorSubcoreMesh(core_axis_name="c", subcore_axis_name="s")

    @functools.partial(
        pl.kernel,
        mesh=mesh,
        out_type=jax.ShapeDtypeStruct((nrows, ncols), jnp.float32),
        scratch_types=[
            pltpu.VMEM((per_w,), jnp.int32),
            pltpu.VMEM((chunk, ncols), jnp.float32),
            pltpu.VMEM((chunk, ncols), jnp.float32),
            pltpu.SemaphoreType.DMA,
            pltpu.SemaphoreType.DMA,
        ],
        compiler_params=pltpu.CompilerParams(needs_layout_passes=False),
    )
    def _gather(idx_hbm, table_hbm, out_hbm, idx_v, buf0, buf1, sem_g, sem_w):
        wid = lax.axis_index("s") * NC + lax.axis_index("c")
        base = wid * per_w
        pltpu.sync_copy(idx_hbm.at[pl.ds(base, per_w)], idx_v)
        bufs = [buf0, buf1]
        g = [None] * nch
        w = [None] * nch
        g[0] = pltpu.async_copy(
            table_hbm.at[idx_v.at[pl.ds(0, chunk)]], bufs[0], sem_g)
        for c in range(nch):
            g[c].wait()
            if c + 1 < nch:
                if c >= 1:
                    w[c - 1].wait()
                g[c + 1] = pltpu.async_copy(
                    table_hbm.at[idx_v.at[pl.ds((c + 1) * chunk, chunk)]],
                    bufs[(c + 1) % 2], sem_g)
            w[c] = pltpu.async_copy(
                bufs[c % 2], out_hbm.at[pl.ds(base + c * chunk, chunk)], sem_w)
        if nch >= 2:
            w[nch - 2].wait()
        w[nch - 1].wait()

    return _gather


def _ffn_body(blk_ref, nb_ref, xs_ref, wg_ref, wu_ref, wd_ref, yp_ref):
    b = pl.program_id(0)

    @pl.when(b < nb_ref[0])
    def _():
        xb = xs_ref[...]
        hg = lax.dot_general(xb, wg_ref[0], (((1,), (1,)), ((), ())),
                             preferred_element_type=jnp.float32)
        hu = lax.dot_general(xb, wu_ref[0], (((1,), (1,)), ((), ())),
                             preferred_element_type=jnp.float32)
        h = hg * lax.logistic(hg) * hu
        yp_ref[...] = lax.dot_general(h, wd_ref[0], (((1,), (1,)), ((), ())),
                                      preferred_element_type=jnp.float32)


def _grouped_ffn(blk, nb, xs, Wg, Wu, Wd):
    grid_spec = pltpu.PrefetchScalarGridSpec(
        num_scalar_prefetch=2,
        grid=(NB,),
        in_specs=[
            pl.BlockSpec((BT, D), lambda b, blk, nb: (b, 0)),
            pl.BlockSpec((1, FF, D), lambda b, blk, nb: (blk[b], 0, 0)),
            pl.BlockSpec((1, FF, D), lambda b, blk, nb: (blk[b], 0, 0)),
            pl.BlockSpec((1, D, FF), lambda b, blk, nb: (blk[b], 0, 0)),
        ],
        out_specs=pl.BlockSpec((BT, D), lambda b, blk, nb: (b, 0)),
    )
    return pl.pallas_call(
        _ffn_body,
        grid_spec=grid_spec,
        out_shape=jax.ShapeDtypeStruct((NP, D), jnp.float32),
    )(blk, nb, xs, Wg, Wu, Wd)


def _combine_body(ypt_ref, scores_ref, y_ref):
    s = jnp.expand_dims(scores_ref[...], -1)
    y_ref[...] = jnp.sum(ypt_ref[...] * s, axis=1)


def _combine(ypt, scores):
    return pl.pallas_call(
        _combine_body,
        grid=(T // BTR,),
        in_specs=[
            pl.BlockSpec((BTR, K, D), lambda t: (t, 0, 0)),
            pl.BlockSpec((BTR, K), lambda t: (t, 0)),
        ],
        out_specs=pl.BlockSpec((BTR, D), lambda t: (t, 0)),
        out_shape=jax.ShapeDtypeStruct((T, D), jnp.float32),
    )(ypt, scores)


def kernel(x, Wr, Wg, Wu, Wd, expert_bias):
    bias2 = expert_bias.reshape(1, E)
    scores, inds, rank, cnt = _router(x, Wr, bias2)
    pos, blk, nb = _finalize(cnt, inds, rank)
    posflat = pos.reshape(TK)
    tok = _make_sc_tok_build()(posflat)
    xs = _make_sc_row_gather(NP, D, 48)(tok, x)
    yp = _grouped_ffn(blk.reshape(NB), nb.reshape(1), xs, Wg, Wu, Wd)
    ypt = _make_sc_row_gather(TK, D, 32)(posflat, yp)
    y = _combine(ypt.reshape(T, K, D), scores)
    return y


# speedup vs baseline: 1.6212x; 1.1046x over previous
import functools

import jax
import jax.numpy as jnp
from jax import lax
from jax.experimental import pallas as pl
from jax.experimental.pallas import tpu as pltpu
from jax.experimental.pallas import tpu_sc as plsc

T = 2048
D = 1024
E = 16
K = 2
FF = 512
TK = T * K

BT = 128
NB = 48
NP = NB * BT

BTR = 256
NTB = T // BTR
NC, NS, L = 2, 16, 16
NW = NC * NS


def _router_body(x_ref, wr_ref, bias_ref, scores_ref, pos_ref, blk_ref,
                 nb_ref, carry_s, inds_s, rank_s):
    t = pl.program_id(0)

    @pl.when(t == 0)
    def _init():
        carry_s[...] = jnp.zeros_like(carry_s)

    @pl.when(t < NTB)
    def _route():
        xb = x_ref[...]
        logits = lax.dot_general(xb, wr_ref[...], (((1,), (1,)), ((), ())),
                                 preferred_element_type=jnp.float32)
        m = jnp.max(logits, axis=-1, keepdims=True)
        p = jnp.exp(logits - m)
        gates = p / jnp.sum(p, axis=-1, keepdims=True)
        g = gates + bias_ref[...]
        iota = lax.broadcasted_iota(jnp.int32, (BTR, E), 1)
        m1 = jnp.max(g, axis=-1, keepdims=True)
        i1 = jnp.min(jnp.where(g == m1, iota, E), axis=-1, keepdims=True)
        g2 = jnp.where(iota == i1, -1e30, g)
        m2 = jnp.max(g2, axis=-1, keepdims=True)
        i2 = jnp.min(jnp.where(g2 == m2, iota, E), axis=-1, keepdims=True)
        denom = m1 + m2 + 1e-20
        s1 = m1 / denom
        s2 = m2 / denom

        a1 = (iota == i1).astype(jnp.float32)
        a2 = (iota == i2).astype(jnp.float32)
        both = a1 + a2
        ri = lax.broadcasted_iota(jnp.int32, (BTR, BTR), 0)
        ci = lax.broadcasted_iota(jnp.int32, (BTR, BTR), 1)
        tril = (ri > ci).astype(jnp.float32)
        cum = lax.dot_general(tril, both, (((1,), (0,)), ((), ())),
                              preferred_element_type=jnp.float32)
        carry = carry_s[...]
        r0 = jnp.sum((cum + carry) * a1, axis=1, keepdims=True)
        r1 = jnp.sum((cum + carry) * a2, axis=1, keepdims=True)
        carry_s[...] = carry + jnp.sum(both, axis=0, keepdims=True)

        scores_ref[...] = jnp.concatenate([s1, s2], axis=1)
        row = pl.multiple_of(t * BTR, BTR)
        inds_s[pl.ds(row, BTR), :] = jnp.concatenate([i1, i2], axis=1)
        rank_s[pl.ds(row, BTR), :] = jnp.concatenate(
            [r0, r1], axis=1).astype(jnp.int32)

    @pl.when(t == NTB)
    def _finalize():
        cnt = carry_s[...].astype(jnp.int32)
        nbe = (cnt + (BT - 1)) // BT
        nbef = nbe.astype(jnp.float32)
        er = lax.broadcasted_iota(jnp.int32, (E, E), 0)
        ec = lax.broadcasted_iota(jnp.int32, (E, E), 1)
        triu = (er < ec).astype(jnp.float32)
        pblk = lax.dot_general(nbef, triu, (((1,), (0,)), ((), ())),
                               preferred_element_type=jnp.float32)
        pstart = pblk * BT

        inds = inds_s[...]
        rank = rank_s[...]
        iota_e = lax.broadcasted_iota(jnp.int32, (T, E), 1)
        iota_k = lax.broadcasted_iota(jnp.int32, (T, K), 1)
        i1 = jnp.sum(jnp.where(iota_k == 0, inds, 0), axis=1, keepdims=True)
        i2 = jnp.sum(jnp.where(iota_k == 1, inds, 0), axis=1, keepdims=True)
        r1 = jnp.sum(jnp.where(iota_k == 0, rank, 0), axis=1, keepdims=True)
        r2 = jnp.sum(jnp.where(iota_k == 1, rank, 0), axis=1, keepdims=True)
        pg1 = jnp.sum(jnp.where(i1 == iota_e, pstart, 0.0), axis=1,
                      keepdims=True)
        pg2 = jnp.sum(jnp.where(i2 == iota_e, pstart, 0.0), axis=1,
                      keepdims=True)
        pos0 = pg1.astype(jnp.int32) + r1
        pos1 = pg2.astype(jnp.int32) + r2
        pos_ref[...] = jnp.concatenate([pos0, pos1], axis=1)

        cb = pblk + nbef
        bf = lax.broadcasted_iota(jnp.int32, (NB, E), 0).astype(jnp.float32)
        blk = jnp.sum((cb <= bf).astype(jnp.int32), axis=1, keepdims=True)
        blk_ref[...] = jnp.minimum(blk, E - 1)
        nb_ref[...] = jnp.sum(nbe, axis=1, keepdims=True)


def _router_finalize(x, Wr, bias2):
    last = NTB - 1
    return pl.pallas_call(
        _router_body,
        grid=(NTB + 1,),
        in_specs=[
            pl.BlockSpec((BTR, D), lambda t: (jnp.minimum(t, last), 0)),
            pl.BlockSpec((E, D), lambda t: (0, 0)),
            pl.BlockSpec((1, E), lambda t: (0, 0)),
        ],
        out_specs=[
            pl.BlockSpec((BTR, K), lambda t: (jnp.minimum(t, last), 0)),
            pl.BlockSpec((T, K), lambda t: (0, 0)),
            pl.BlockSpec((NB, 1), lambda t: (0, 0)),
            pl.BlockSpec((1, 1), lambda t: (0, 0)),
        ],
        out_shape=[
            jax.ShapeDtypeStruct((T, K), jnp.float32),
            jax.ShapeDtypeStruct((T, K), jnp.int32),
            jax.ShapeDtypeStruct((NB, 1), jnp.int32),
            jax.ShapeDtypeStruct((1, 1), jnp.int32),
        ],
        scratch_shapes=[
            pltpu.VMEM((1, E), jnp.float32),
            pltpu.VMEM((T, K), jnp.int32),
            pltpu.VMEM((T, K), jnp.int32),
        ],
    )(x, Wr, bias2)


@functools.cache
def _make_sc_scatter_xs():
    per_w = T // NW
    mesh = plsc.VectorSubcoreMesh(core_axis_name="c", subcore_axis_name="s")

    @functools.partial(
        pl.kernel,
        mesh=mesh,
        out_type=jax.ShapeDtypeStruct((NP, D), jnp.float32),
        scratch_types=[
            pltpu.VMEM((K * per_w,), jnp.int32),
            pltpu.VMEM((per_w,), jnp.int32),
            pltpu.VMEM((per_w,), jnp.int32),
            pltpu.VMEM((per_w, D), jnp.float32),
            pltpu.SemaphoreType.DMA,
        ],
        compiler_params=pltpu.CompilerParams(needs_layout_passes=False),
    )
    def _scatter(pos_hbm, x_hbm, xs_hbm, idxall_v, idx0_v, idx1_v, rows_v,
                 sem):
        wid = lax.axis_index("s") * NC + lax.axis_index("c")
        base = wid * per_w
        pltpu.sync_copy(pos_hbm.at[pl.ds(base * K, per_w * K)], idxall_v)
        pltpu.sync_copy(x_hbm.at[pl.ds(base, per_w)], rows_v)
        iota = lax.broadcasted_iota(jnp.int32, (L,), 0)
        for c in range(per_w // L):
            lanes = iota * K + c * (L * K)
            idx0_v[pl.ds(c * L, L)] = plsc.load_gather(idxall_v, [lanes])
            idx1_v[pl.ds(c * L, L)] = plsc.load_gather(idxall_v, [lanes + 1])
        c0 = pltpu.async_copy(rows_v, xs_hbm.at[idx0_v], sem)
        c1 = pltpu.async_copy(rows_v, xs_hbm.at[idx1_v], sem)
        c0.wait()
        c1.wait()

    return _scatter


@functools.cache
def _make_sc_row_gather(nrows, ncols, chunk):
    per_w = nrows // NW
    nch = per_w // chunk
    mesh = plsc.VectorSubcoreMesh(core_axis_name="c", subcore_axis_name="s")

    @functools.partial(
        pl.kernel,
        mesh=mesh,
        out_type=jax.ShapeDtypeStruct((nrows, ncols), jnp.float32),
        scratch_types=[
            pltpu.VMEM((per_w,), jnp.int32),
            pltpu.VMEM((chunk, ncols), jnp.float32),
            pltpu.VMEM((chunk, ncols), jnp.float32),
            pltpu.SemaphoreType.DMA,
            pltpu.SemaphoreType.DMA,
        ],
        compiler_params=pltpu.CompilerParams(needs_layout_passes=False),
    )
    def _gather(idx_hbm, table_hbm, out_hbm, idx_v, buf0, buf1, sem_g, sem_w):
        wid = lax.axis_index("s") * NC + lax.axis_index("c")
        base = wid * per_w
        pltpu.sync_copy(idx_hbm.at[pl.ds(base, per_w)], idx_v)
        bufs = [buf0, buf1]
        g = [None] * nch
        w = [None] * nch
        g[0] = pltpu.async_copy(
            table_hbm.at[idx_v.at[pl.ds(0, chunk)]], bufs[0], sem_g)
        for c in range(nch):
            g[c].wait()
            if c + 1 < nch:
                if c >= 1:
                    w[c - 1].wait()
                g[c + 1] = pltpu.async_copy(
                    table_hbm.at[idx_v.at[pl.ds((c + 1) * chunk, chunk)]],
                    bufs[(c + 1) % 2], sem_g)
            w[c] = pltpu.async_copy(
                bufs[c % 2], out_hbm.at[pl.ds(base + c * chunk, chunk)], sem_w)
        if nch >= 2:
            w[nch - 2].wait()
        w[nch - 1].wait()

    return _gather


def _ffn_body(blk_ref, nb_ref, xs_ref, wg_ref, wu_ref, wd_ref, yp_ref):
    b = pl.program_id(0)

    @pl.when(b < nb_ref[0])
    def _():
        xb = xs_ref[...]
        hg = lax.dot_general(xb, wg_ref[0], (((1,), (1,)), ((), ())),
                             preferred_element_type=jnp.float32)
        hu = lax.dot_general(xb, wu_ref[0], (((1,), (1,)), ((), ())),
                             preferred_element_type=jnp.float32)
        h = hg * lax.logistic(hg) * hu
        yp_ref[...] = lax.dot_general(h, wd_ref[0], (((1,), (1,)), ((), ())),
                                      preferred_element_type=jnp.float32)


def _grouped_ffn(blk, nb, xs, Wg, Wu, Wd):
    grid_spec = pltpu.PrefetchScalarGridSpec(
        num_scalar_prefetch=2,
        grid=(NB,),
        in_specs=[
            pl.BlockSpec((BT, D), lambda b, blk, nb: (b, 0)),
            pl.BlockSpec((1, FF, D), lambda b, blk, nb: (blk[b], 0, 0)),
            pl.BlockSpec((1, FF, D), lambda b, blk, nb: (blk[b], 0, 0)),
            pl.BlockSpec((1, D, FF), lambda b, blk, nb: (blk[b], 0, 0)),
        ],
        out_specs=pl.BlockSpec((BT, D), lambda b, blk, nb: (b, 0)),
    )
    return pl.pallas_call(
        _ffn_body,
        grid_spec=grid_spec,
        out_shape=jax.ShapeDtypeStruct((NP, D), jnp.float32),
    )(blk, nb, xs, Wg, Wu, Wd)


def _combine_body(ypt_ref, scores_ref, y_ref):
    s = jnp.expand_dims(scores_ref[...], -1)
    y_ref[...] = jnp.sum(ypt_ref[...] * s, axis=1)


def _combine(ypt, scores):
    return pl.pallas_call(
        _combine_body,
        grid=(T // BTR,),
        in_specs=[
            pl.BlockSpec((BTR, K, D), lambda t: (t, 0, 0)),
            pl.BlockSpec((BTR, K), lambda t: (t, 0)),
        ],
        out_specs=pl.BlockSpec((BTR, D), lambda t: (t, 0)),
        out_shape=jax.ShapeDtypeStruct((T, D), jnp.float32),
    )(ypt, scores)


def kernel(x, Wr, Wg, Wu, Wd, expert_bias):
    bias2 = expert_bias.reshape(1, E)
    scores, pos, blk, nb = _router_finalize(x, Wr, bias2)
    posflat = pos.reshape(TK)
    xs = _make_sc_scatter_xs()(posflat, x)
    yp = _grouped_ffn(blk.reshape(NB), nb.reshape(1), xs, Wg, Wu, Wd)
    ypt = _make_sc_row_gather(TK, D, 32)(posflat, yp)
    y = _combine(ypt.reshape(T, K, D), scores)
    return y


# speedup vs baseline: 2.2089x; 1.3625x over previous
import functools

import jax
import jax.numpy as jnp
from jax import lax
from jax.experimental import pallas as pl
from jax.experimental.pallas import tpu as pltpu
from jax.experimental.pallas import tpu_sc as plsc

T = 2048
D = 1024
E = 16
K = 2
FF = 512
TK = T * K

BT = 256
NB = 24
NP = NB * BT

BTR = 256
NTB = T // BTR
NC, NS, L = 2, 16, 16
NW = NC * NS


def _router_body(x_ref, wr_ref, bias_ref, scores_ref, pos_ref, blk_ref,
                 nb_ref, carry_s, inds_s, rank_s):
    t = pl.program_id(0)

    @pl.when(t == 0)
    def _init():
        carry_s[...] = jnp.zeros_like(carry_s)

    @pl.when(t < NTB)
    def _route():
        xb = x_ref[...]
        logits = lax.dot_general(xb, wr_ref[...], (((1,), (1,)), ((), ())),
                                 preferred_element_type=jnp.float32)
        m = jnp.max(logits, axis=-1, keepdims=True)
        p = jnp.exp(logits - m)
        gates = p / jnp.sum(p, axis=-1, keepdims=True)
        g = gates + bias_ref[...]
        iota = lax.broadcasted_iota(jnp.int32, (BTR, E), 1)
        m1 = jnp.max(g, axis=-1, keepdims=True)
        i1 = jnp.min(jnp.where(g == m1, iota, E), axis=-1, keepdims=True)
        g2 = jnp.where(iota == i1, -1e30, g)
        m2 = jnp.max(g2, axis=-1, keepdims=True)
        i2 = jnp.min(jnp.where(g2 == m2, iota, E), axis=-1, keepdims=True)
        denom = m1 + m2 + 1e-20
        s1 = m1 / denom
        s2 = m2 / denom

        a1 = (iota == i1).astype(jnp.float32)
        a2 = (iota == i2).astype(jnp.float32)
        both = a1 + a2
        ri = lax.broadcasted_iota(jnp.int32, (BTR, BTR), 0)
        ci = lax.broadcasted_iota(jnp.int32, (BTR, BTR), 1)
        tril = (ri > ci).astype(jnp.float32)
        cum = lax.dot_general(tril, both, (((1,), (0,)), ((), ())),
                              preferred_element_type=jnp.float32)
        carry = carry_s[...]
        r0 = jnp.sum((cum + carry) * a1, axis=1, keepdims=True)
        r1 = jnp.sum((cum + carry) * a2, axis=1, keepdims=True)
        carry_s[...] = carry + jnp.sum(both, axis=0, keepdims=True)

        scores_ref[...] = jnp.concatenate([s1, s2], axis=1)
        row = pl.multiple_of(t * BTR, BTR)
        inds_s[pl.ds(row, BTR), :] = jnp.concatenate([i1, i2], axis=1)
        rank_s[pl.ds(row, BTR), :] = jnp.concatenate(
            [r0, r1], axis=1).astype(jnp.int32)

    @pl.when(t == NTB)
    def _finalize():
        cnt = carry_s[...].astype(jnp.int32)
        nbe = (cnt + (BT - 1)) // BT
        nbef = nbe.astype(jnp.float32)
        er = lax.broadcasted_iota(jnp.int32, (E, E), 0)
        ec = lax.broadcasted_iota(jnp.int32, (E, E), 1)
        triu = (er < ec).astype(jnp.float32)
        pblk = lax.dot_general(nbef, triu, (((1,), (0,)), ((), ())),
                               preferred_element_type=jnp.float32)
        pstart = pblk * BT

        inds = inds_s[...]
        rank = rank_s[...]
        iota_e = lax.broadcasted_iota(jnp.int32, (T, E), 1)
        iota_k = lax.broadcasted_iota(jnp.int32, (T, K), 1)
        i1 = jnp.sum(jnp.where(iota_k == 0, inds, 0), axis=1, keepdims=True)
        i2 = jnp.sum(jnp.where(iota_k == 1, inds, 0), axis=1, keepdims=True)
        r1 = jnp.sum(jnp.where(iota_k == 0, rank, 0), axis=1, keepdims=True)
        r2 = jnp.sum(jnp.where(iota_k == 1, rank, 0), axis=1, keepdims=True)
        pg1 = jnp.sum(jnp.where(i1 == iota_e, pstart, 0.0), axis=1,
                      keepdims=True)
        pg2 = jnp.sum(jnp.where(i2 == iota_e, pstart, 0.0), axis=1,
                      keepdims=True)
        pos0 = pg1.astype(jnp.int32) + r1
        pos1 = pg2.astype(jnp.int32) + r2
        pos_ref[...] = jnp.concatenate([pos0, pos1], axis=1)

        cb = pblk + nbef
        bf = lax.broadcasted_iota(jnp.int32, (NB, E), 0).astype(jnp.float32)
        blk = jnp.sum((cb <= bf).astype(jnp.int32), axis=1, keepdims=True)
        blk_ref[...] = jnp.minimum(blk, E - 1)
        nb_ref[...] = jnp.sum(nbe, axis=1, keepdims=True)


def _router_finalize(x, Wr, bias2):
    last = NTB - 1
    return pl.pallas_call(
        _router_body,
        grid=(NTB + 1,),
        in_specs=[
            pl.BlockSpec((BTR, D), lambda t: (jnp.minimum(t, last), 0)),
            pl.BlockSpec((E, D), lambda t: (0, 0)),
            pl.BlockSpec((1, E), lambda t: (0, 0)),
        ],
        out_specs=[
            pl.BlockSpec((BTR, K), lambda t: (jnp.minimum(t, last), 0)),
            pl.BlockSpec((T, K), lambda t: (0, 0)),
            pl.BlockSpec((NB, 1), lambda t: (0, 0)),
            pl.BlockSpec((1, 1), lambda t: (0, 0)),
        ],
        out_shape=[
            jax.ShapeDtypeStruct((T, K), jnp.float32),
            jax.ShapeDtypeStruct((T, K), jnp.int32),
            jax.ShapeDtypeStruct((NB, 1), jnp.int32),
            jax.ShapeDtypeStruct((1, 1), jnp.int32),
        ],
        scratch_shapes=[
            pltpu.VMEM((1, E), jnp.float32),
            pltpu.VMEM((T, K), jnp.int32),
            pltpu.VMEM((T, K), jnp.int32),
        ],
    )(x, Wr, bias2)


@functools.cache
def _make_sc_scatter_xs():
    per_w = T // NW
    mesh = plsc.VectorSubcoreMesh(core_axis_name="c", subcore_axis_name="s")

    @functools.partial(
        pl.kernel,
        mesh=mesh,
        out_type=jax.ShapeDtypeStruct((NP, D), jnp.float32),
        scratch_types=[
            pltpu.VMEM((K * per_w,), jnp.int32),
            pltpu.VMEM((per_w,), jnp.int32),
            pltpu.VMEM((per_w,), jnp.int32),
            pltpu.VMEM((per_w, D), jnp.float32),
            pltpu.SemaphoreType.DMA,
        ],
        compiler_params=pltpu.CompilerParams(needs_layout_passes=False),
    )
    def _scatter(pos_hbm, x_hbm, xs_hbm, idxall_v, idx0_v, idx1_v, rows_v,
                 sem):
        wid = lax.axis_index("s") * NC + lax.axis_index("c")
        base = wid * per_w
        pltpu.sync_copy(pos_hbm.at[pl.ds(base * K, per_w * K)], idxall_v)
        pltpu.sync_copy(x_hbm.at[pl.ds(base, per_w)], rows_v)
        iota = lax.broadcasted_iota(jnp.int32, (L,), 0)
        for c in range(per_w // L):
            lanes = iota * K + c * (L * K)
            idx0_v[pl.ds(c * L, L)] = plsc.load_gather(idxall_v, [lanes])
            idx1_v[pl.ds(c * L, L)] = plsc.load_gather(idxall_v, [lanes + 1])
        c0 = pltpu.async_copy(rows_v, xs_hbm.at[idx0_v], sem)
        c1 = pltpu.async_copy(rows_v, xs_hbm.at[idx1_v], sem)
        c0.wait()
        c1.wait()

    return _scatter


@functools.cache
def _make_sc_row_gather(nrows, ncols, chunk):
    per_w = nrows // NW
    nch = per_w // chunk
    mesh = plsc.VectorSubcoreMesh(core_axis_name="c", subcore_axis_name="s")

    @functools.partial(
        pl.kernel,
        mesh=mesh,
        out_type=jax.ShapeDtypeStruct((nrows, ncols), jnp.float32),
        scratch_types=[
            pltpu.VMEM((per_w,), jnp.int32),
            pltpu.VMEM((chunk, ncols), jnp.float32),
            pltpu.VMEM((chunk, ncols), jnp.float32),
            pltpu.SemaphoreType.DMA,
            pltpu.SemaphoreType.DMA,
        ],
        compiler_params=pltpu.CompilerParams(needs_layout_passes=False),
    )
    def _gather(idx_hbm, table_hbm, out_hbm, idx_v, buf0, buf1, sem_g, sem_w):
        wid = lax.axis_index("s") * NC + lax.axis_index("c")
        base = wid * per_w
        pltpu.sync_copy(idx_hbm.at[pl.ds(base, per_w)], idx_v)
        bufs = [buf0, buf1]
        g = [None] * nch
        w = [None] * nch
        g[0] = pltpu.async_copy(
            table_hbm.at[idx_v.at[pl.ds(0, chunk)]], bufs[0], sem_g)
        for c in range(nch):
            g[c].wait()
            if c + 1 < nch:
                if c >= 1:
                    w[c - 1].wait()
                g[c + 1] = pltpu.async_copy(
                    table_hbm.at[idx_v.at[pl.ds((c + 1) * chunk, chunk)]],
                    bufs[(c + 1) % 2], sem_g)
            w[c] = pltpu.async_copy(
                bufs[c % 2], out_hbm.at[pl.ds(base + c * chunk, chunk)], sem_w)
        if nch >= 2:
            w[nch - 2].wait()
        w[nch - 1].wait()

    return _gather


def _ffn_body(blk_ref, nb_ref, xs_ref, wg_ref, wu_ref, wd_ref, yp_ref):
    b = pl.program_id(0)

    @pl.when(b < nb_ref[0])
    def _():
        xb = xs_ref[...]
        hg = lax.dot_general(xb, wg_ref[0], (((1,), (1,)), ((), ())),
                             preferred_element_type=jnp.float32)
        hu = lax.dot_general(xb, wu_ref[0], (((1,), (1,)), ((), ())),
                             preferred_element_type=jnp.float32)
        h = hg * lax.logistic(hg) * hu
        yp_ref[...] = lax.dot_general(h, wd_ref[0], (((1,), (1,)), ((), ())),
                                      preferred_element_type=jnp.float32)


def _grouped_ffn(blk, nb, xs, Wg, Wu, Wd):
    grid_spec = pltpu.PrefetchScalarGridSpec(
        num_scalar_prefetch=2,
        grid=(NB,),
        in_specs=[
            pl.BlockSpec((BT, D), lambda b, blk, nb: (b, 0)),
            pl.BlockSpec((1, FF, D), lambda b, blk, nb: (blk[b], 0, 0)),
            pl.BlockSpec((1, FF, D), lambda b, blk, nb: (blk[b], 0, 0)),
            pl.BlockSpec((1, D, FF), lambda b, blk, nb: (blk[b], 0, 0)),
        ],
        out_specs=pl.BlockSpec((BT, D), lambda b, blk, nb: (b, 0)),
    )
    return pl.pallas_call(
        _ffn_body,
        grid_spec=grid_spec,
        out_shape=jax.ShapeDtypeStruct((NP, D), jnp.float32),
    )(blk, nb, xs, Wg, Wu, Wd)


def _combine_body(ypt_ref, scores_ref, y_ref):
    rows = ypt_ref[...].reshape(BTR, K, D)
    s = jnp.expand_dims(scores_ref[...], -1)
    y_ref[...] = jnp.sum(rows * s, axis=1)


def _combine(ypt, scores):
    return pl.pallas_call(
        _combine_body,
        grid=(T // BTR,),
        in_specs=[
            pl.BlockSpec((K * BTR, D), lambda t: (t, 0)),
            pl.BlockSpec((BTR, K), lambda t: (t, 0)),
        ],
        out_specs=pl.BlockSpec((BTR, D), lambda t: (t, 0)),
        out_shape=jax.ShapeDtypeStruct((T, D), jnp.float32),
    )(ypt, scores)


def kernel(x, Wr, Wg, Wu, Wd, expert_bias):
    bias2 = expert_bias.reshape(1, E)
    scores, pos, blk, nb = _router_finalize(x, Wr, bias2)
    posflat = pos.reshape(TK)
    xs = _make_sc_scatter_xs()(posflat, x)
    yp = _grouped_ffn(blk.reshape(NB), nb.reshape(1), xs, Wg, Wu, Wd)
    ypt = _make_sc_row_gather(TK, D, 32)(posflat, yp)
    y = _combine(ypt, scores)
    return y


# speedup vs baseline: 2.4190x; 1.0951x over previous
import functools

import jax
import jax.numpy as jnp
from jax import lax
from jax.experimental import pallas as pl
from jax.experimental.pallas import tpu as pltpu
from jax.experimental.pallas import tpu_sc as plsc

T = 2048
D = 1024
E = 16
K = 2
FF = 512
TK = T * K

BT = 256
NB = 24
NP = NB * BT

BTR = 256
NTB = T // BTR
NC, NS, L = 2, 16, 16
NW = NC * NS


def _router_body(x_ref, wr_ref, bias_ref, scores_ref, pos_ref, blk_ref,
                 nb_ref, carry_s, inds_s, rank_s):
    t = pl.program_id(0)

    @pl.when(t == 0)
    def _init():
        carry_s[...] = jnp.zeros_like(carry_s)

    @pl.when(t < NTB)
    def _route():
        xb = x_ref[...]
        logits = lax.dot_general(xb, wr_ref[...], (((1,), (1,)), ((), ())),
                                 preferred_element_type=jnp.float32)
        m = jnp.max(logits, axis=-1, keepdims=True)
        p = jnp.exp(logits - m)
        gates = p / jnp.sum(p, axis=-1, keepdims=True)
        g = gates + bias_ref[...]
        iota = lax.broadcasted_iota(jnp.int32, (BTR, E), 1)
        m1 = jnp.max(g, axis=-1, keepdims=True)
        i1 = jnp.min(jnp.where(g == m1, iota, E), axis=-1, keepdims=True)
        g2 = jnp.where(iota == i1, -1e30, g)
        m2 = jnp.max(g2, axis=-1, keepdims=True)
        i2 = jnp.min(jnp.where(g2 == m2, iota, E), axis=-1, keepdims=True)
        denom = m1 + m2 + 1e-20
        s1 = m1 / denom
        s2 = m2 / denom

        a1 = (iota == i1).astype(jnp.float32)
        a2 = (iota == i2).astype(jnp.float32)
        both = a1 + a2
        ri = lax.broadcasted_iota(jnp.int32, (BTR, BTR), 0)
        ci = lax.broadcasted_iota(jnp.int32, (BTR, BTR), 1)
        tril = (ri > ci).astype(jnp.float32)
        cum = lax.dot_general(tril, both, (((1,), (0,)), ((), ())),
                              preferred_element_type=jnp.float32)
        carry = carry_s[...]
        r0 = jnp.sum((cum + carry) * a1, axis=1, keepdims=True)
        r1 = jnp.sum((cum + carry) * a2, axis=1, keepdims=True)
        carry_s[...] = carry + jnp.sum(both, axis=0, keepdims=True)

        scores_ref[...] = jnp.concatenate([s1, s2], axis=1)
        row = pl.multiple_of(t * BTR, BTR)
        inds_s[pl.ds(row, BTR), :] = jnp.concatenate([i1, i2], axis=1)
        rank_s[pl.ds(row, BTR), :] = jnp.concatenate(
            [r0, r1], axis=1).astype(jnp.int32)

    @pl.when(t == NTB)
    def _finalize():
        cnt = carry_s[...].astype(jnp.int32)
        nbe = (cnt + (BT - 1)) // BT
        nbef = nbe.astype(jnp.float32)
        er = lax.broadcasted_iota(jnp.int32, (E, E), 0)
        ec = lax.broadcasted_iota(jnp.int32, (E, E), 1)
        triu = (er < ec).astype(jnp.float32)
        pblk = lax.dot_general(nbef, triu, (((1,), (0,)), ((), ())),
                               preferred_element_type=jnp.float32)
        pstart = pblk * BT

        inds = inds_s[...]
        rank = rank_s[...]
        iota_e = lax.broadcasted_iota(jnp.int32, (T, E), 1)
        iota_k = lax.broadcasted_iota(jnp.int32, (T, K), 1)
        i1 = jnp.sum(jnp.where(iota_k == 0, inds, 0), axis=1, keepdims=True)
        i2 = jnp.sum(jnp.where(iota_k == 1, inds, 0), axis=1, keepdims=True)
        r1 = jnp.sum(jnp.where(iota_k == 0, rank, 0), axis=1, keepdims=True)
        r2 = jnp.sum(jnp.where(iota_k == 1, rank, 0), axis=1, keepdims=True)
        pg1 = jnp.sum(jnp.where(i1 == iota_e, pstart, 0.0), axis=1,
                      keepdims=True)
        pg2 = jnp.sum(jnp.where(i2 == iota_e, pstart, 0.0), axis=1,
                      keepdims=True)
        pos0 = pg1.astype(jnp.int32) + r1
        pos1 = pg2.astype(jnp.int32) + r2
        pos_ref[...] = jnp.concatenate([pos0, pos1], axis=1)

        cb = pblk + nbef
        bf = lax.broadcasted_iota(jnp.int32, (NB, E), 0).astype(jnp.float32)
        blk = jnp.sum((cb <= bf).astype(jnp.int32), axis=1, keepdims=True)
        blk_ref[...] = jnp.minimum(blk, E - 1)
        nb_ref[...] = jnp.sum(nbe, axis=1, keepdims=True)


def _router_finalize(x, Wr, bias2):
    last = NTB - 1
    return pl.pallas_call(
        _router_body,
        grid=(NTB + 1,),
        in_specs=[
            pl.BlockSpec((BTR, D), lambda t: (jnp.minimum(t, last), 0)),
            pl.BlockSpec((E, D), lambda t: (0, 0)),
            pl.BlockSpec((1, E), lambda t: (0, 0)),
        ],
        out_specs=[
            pl.BlockSpec((BTR, K), lambda t: (jnp.minimum(t, last), 0)),
            pl.BlockSpec((T, K), lambda t: (0, 0)),
            pl.BlockSpec((NB, 1), lambda t: (0, 0)),
            pl.BlockSpec((1, 1), lambda t: (0, 0)),
        ],
        out_shape=[
            jax.ShapeDtypeStruct((T, K), jnp.float32),
            jax.ShapeDtypeStruct((T, K), jnp.int32),
            jax.ShapeDtypeStruct((NB, 1), jnp.int32),
            jax.ShapeDtypeStruct((1, 1), jnp.int32),
        ],
        scratch_shapes=[
            pltpu.VMEM((1, E), jnp.float32),
            pltpu.VMEM((T, K), jnp.int32),
            pltpu.VMEM((T, K), jnp.int32),
        ],
    )(x, Wr, bias2)


@functools.cache
def _make_sc_scatter_xs():
    per_w = T // NW
    mesh = plsc.VectorSubcoreMesh(core_axis_name="c", subcore_axis_name="s")

    @functools.partial(
        pl.kernel,
        mesh=mesh,
        out_type=jax.ShapeDtypeStruct((NP, D), jnp.float32),
        scratch_types=[
            pltpu.VMEM((K * per_w,), jnp.int32),
            pltpu.VMEM((per_w,), jnp.int32),
            pltpu.VMEM((per_w,), jnp.int32),
            pltpu.VMEM((per_w, D), jnp.float32),
            pltpu.SemaphoreType.DMA,
        ],
        compiler_params=pltpu.CompilerParams(needs_layout_passes=False),
    )
    def _scatter(pos_hbm, x_hbm, xs_hbm, idxall_v, idx0_v, idx1_v, rows_v,
                 sem):
        wid = lax.axis_index("s") * NC + lax.axis_index("c")
        base = wid * per_w
        pltpu.sync_copy(pos_hbm.at[pl.ds(base * K, per_w * K)], idxall_v)
        pltpu.sync_copy(x_hbm.at[pl.ds(base, per_w)], rows_v)
        iota = lax.broadcasted_iota(jnp.int32, (L,), 0)
        for c in range(per_w // L):
            lanes = iota * K + c * (L * K)
            idx0_v[pl.ds(c * L, L)] = plsc.load_gather(idxall_v, [lanes])
            idx1_v[pl.ds(c * L, L)] = plsc.load_gather(idxall_v, [lanes + 1])
        c0 = pltpu.async_copy(rows_v, xs_hbm.at[idx0_v], sem)
        c1 = pltpu.async_copy(rows_v, xs_hbm.at[idx1_v], sem)
        c0.wait()
        c1.wait()

    return _scatter


@functools.cache
def _make_sc_row_gather(nrows, ncols, chunk):
    per_w = nrows // NW
    nch = per_w // chunk
    mesh = plsc.VectorSubcoreMesh(core_axis_name="c", subcore_axis_name="s")

    @functools.partial(
        pl.kernel,
        mesh=mesh,
        out_type=jax.ShapeDtypeStruct((nrows, ncols), jnp.float32),
        scratch_types=[
            pltpu.VMEM((per_w,), jnp.int32),
            pltpu.VMEM((chunk, ncols), jnp.float32),
            pltpu.VMEM((chunk, ncols), jnp.float32),
            pltpu.SemaphoreType.DMA,
            pltpu.SemaphoreType.DMA,
        ],
        compiler_params=pltpu.CompilerParams(needs_layout_passes=False),
    )
    def _gather(idx_hbm, table_hbm, out_hbm, idx_v, buf0, buf1, sem_g, sem_w):
        wid = lax.axis_index("s") * NC + lax.axis_index("c")
        base = wid * per_w
        pltpu.sync_copy(idx_hbm.at[pl.ds(base, per_w)], idx_v)
        bufs = [buf0, buf1]
        g = [None] * nch
        w = [None] * nch
        g[0] = pltpu.async_copy(
            table_hbm.at[idx_v.at[pl.ds(0, chunk)]], bufs[0], sem_g)
        for c in range(nch):
            g[c].wait()
            if c + 1 < nch:
                if c >= 1:
                    w[c - 1].wait()
                g[c + 1] = pltpu.async_copy(
                    table_hbm.at[idx_v.at[pl.ds((c + 1) * chunk, chunk)]],
                    bufs[(c + 1) % 2], sem_g)
            w[c] = pltpu.async_copy(
                bufs[c % 2], out_hbm.at[pl.ds(base + c * chunk, chunk)], sem_w)
        if nch >= 2:
            w[nch - 2].wait()
        w[nch - 1].wait()

    return _gather


def _ffn_body(blk_ref, nb_ref, xs_ref, wg_ref, wu_ref, wd_ref, yp_ref):
    b = pl.program_id(0)

    @pl.when(b < nb_ref[0])
    def _():
        xb = xs_ref[...]
        hg = lax.dot_general(xb, wg_ref[0], (((1,), (1,)), ((), ())),
                             preferred_element_type=jnp.float32)
        hu = lax.dot_general(xb, wu_ref[0], (((1,), (1,)), ((), ())),
                             preferred_element_type=jnp.float32)
        h = hg * lax.logistic(hg) * hu
        yp_ref[...] = lax.dot_general(h, wd_ref[0], (((1,), (1,)), ((), ())),
                                      preferred_element_type=jnp.float32)


def _grouped_ffn(blk, nb, xs, Wg, Wu, Wd):
    grid_spec = pltpu.PrefetchScalarGridSpec(
        num_scalar_prefetch=2,
        grid=(NB,),
        in_specs=[
            pl.BlockSpec((BT, D), lambda b, blk, nb: (b, 0)),
            pl.BlockSpec((1, FF, D), lambda b, blk, nb: (blk[b], 0, 0)),
            pl.BlockSpec((1, FF, D), lambda b, blk, nb: (blk[b], 0, 0)),
            pl.BlockSpec((1, D, FF), lambda b, blk, nb: (blk[b], 0, 0)),
        ],
        out_specs=pl.BlockSpec((BT, D), lambda b, blk, nb: (b, 0)),
    )
    return pl.pallas_call(
        _ffn_body,
        grid_spec=grid_spec,
        out_shape=jax.ShapeDtypeStruct((NP, D), jnp.float32),
    )(blk, nb, xs, Wg, Wu, Wd)


def _combine_body(ypt_ref, scores_ref, y_ref):
    s = scores_ref[...]
    iota_k = lax.broadcasted_iota(jnp.int32, (BTR, K), 1)
    s0 = jnp.sum(jnp.where(iota_k == 0, s, 0.0), axis=1, keepdims=True)
    s1 = jnp.sum(jnp.where(iota_k == 1, s, 0.0), axis=1, keepdims=True)
    ri = lax.broadcasted_iota(jnp.int32, (BTR, K * BTR), 0)
    ci = lax.broadcasted_iota(jnp.int32, (BTR, K * BTR), 1)
    a = (jnp.where(ci == 2 * ri, s0, 0.0)
         + jnp.where(ci == 2 * ri + 1, s1, 0.0))
    y_ref[...] = lax.dot_general(a, ypt_ref[...], (((1,), (0,)), ((), ())),
                                 preferred_element_type=jnp.float32)


def _combine(ypt, scores):
    return pl.pallas_call(
        _combine_body,
        grid=(T // BTR,),
        in_specs=[
            pl.BlockSpec((K * BTR, D), lambda t: (t, 0)),
            pl.BlockSpec((BTR, K), lambda t: (t, 0)),
        ],
        out_specs=pl.BlockSpec((BTR, D), lambda t: (t, 0)),
        out_shape=jax.ShapeDtypeStruct((T, D), jnp.float32),
    )(ypt, scores)


def kernel(x, Wr, Wg, Wu, Wd, expert_bias):
    bias2 = expert_bias.reshape(1, E)
    scores, pos, blk, nb = _router_finalize(x, Wr, bias2)
    posflat = pos.reshape(TK)
    xs = _make_sc_scatter_xs()(posflat, x)
    yp = _grouped_ffn(blk.reshape(NB), nb.reshape(1), xs, Wg, Wu, Wd)
    ypt = _make_sc_row_gather(TK, D, 32)(posflat, yp)
    y = _combine(ypt, scores)
    return y


# speedup vs baseline: 2.4708x; 1.0214x over previous
import functools

import jax
import jax.numpy as jnp
from jax import lax
from jax.experimental import pallas as pl
from jax.experimental.pallas import tpu as pltpu
from jax.experimental.pallas import tpu_sc as plsc

T = 2048
D = 1024
E = 16
K = 2
FF = 512
TK = T * K

BT = 256
NB = 24
NP = NB * BT

BTR = 512
NTB = T // BTR
BTC = 256
NC, NS, L = 2, 16, 16
NW = NC * NS


def _router_body(x_ref, wr_ref, bias_ref, scores_ref, pos_ref, blk_ref,
                 nb_ref, carry_s, inds_s, rank_s):
    t = pl.program_id(0)

    @pl.when(t == 0)
    def _init():
        carry_s[...] = jnp.zeros_like(carry_s)

    @pl.when(t < NTB)
    def _route():
        xb = x_ref[...]
        logits = lax.dot_general(xb, wr_ref[...], (((1,), (1,)), ((), ())),
                                 preferred_element_type=jnp.float32)
        m = jnp.max(logits, axis=-1, keepdims=True)
        p = jnp.exp(logits - m)
        gates = p / jnp.sum(p, axis=-1, keepdims=True)
        g = gates + bias_ref[...]
        iota = lax.broadcasted_iota(jnp.int32, (BTR, E), 1)
        m1 = jnp.max(g, axis=-1, keepdims=True)
        i1 = jnp.min(jnp.where(g == m1, iota, E), axis=-1, keepdims=True)
        g2 = jnp.where(iota == i1, -1e30, g)
        m2 = jnp.max(g2, axis=-1, keepdims=True)
        i2 = jnp.min(jnp.where(g2 == m2, iota, E), axis=-1, keepdims=True)
        denom = m1 + m2 + 1e-20
        s1 = m1 / denom
        s2 = m2 / denom

        a1 = (iota == i1).astype(jnp.float32)
        a2 = (iota == i2).astype(jnp.float32)
        both = a1 + a2
        ri = lax.broadcasted_iota(jnp.int32, (BTR, BTR), 0)
        ci = lax.broadcasted_iota(jnp.int32, (BTR, BTR), 1)
        tril = (ri > ci).astype(jnp.float32)
        cum = lax.dot_general(tril, both, (((1,), (0,)), ((), ())),
                              preferred_element_type=jnp.float32)
        carry = carry_s[...]
        r0 = jnp.sum((cum + carry) * a1, axis=1, keepdims=True)
        r1 = jnp.sum((cum + carry) * a2, axis=1, keepdims=True)
        carry_s[...] = carry + jnp.sum(both, axis=0, keepdims=True)

        scores_ref[...] = jnp.concatenate([s1, s2], axis=1)
        row = pl.multiple_of(t * BTR, BTR)
        inds_s[pl.ds(row, BTR), :] = jnp.concatenate([i1, i2], axis=1)
        rank_s[pl.ds(row, BTR), :] = jnp.concatenate(
            [r0, r1], axis=1).astype(jnp.int32)

    @pl.when(t == NTB)
    def _finalize():
        cnt = carry_s[...].astype(jnp.int32)
        nbe = (cnt + (BT - 1)) // BT
        nbef = nbe.astype(jnp.float32)
        er = lax.broadcasted_iota(jnp.int32, (E, E), 0)
        ec = lax.broadcasted_iota(jnp.int32, (E, E), 1)
        triu = (er < ec).astype(jnp.float32)
        pblk = lax.dot_general(nbef, triu, (((1,), (0,)), ((), ())),
                               preferred_element_type=jnp.float32)
        pstart = pblk * BT

        inds = inds_s[...]
        rank = rank_s[...]
        iota_e = lax.broadcasted_iota(jnp.int32, (T, E), 1)
        iota_k = lax.broadcasted_iota(jnp.int32, (T, K), 1)
        i1 = jnp.sum(jnp.where(iota_k == 0, inds, 0), axis=1, keepdims=True)
        i2 = jnp.sum(jnp.where(iota_k == 1, inds, 0), axis=1, keepdims=True)
        r1 = jnp.sum(jnp.where(iota_k == 0, rank, 0), axis=1, keepdims=True)
        r2 = jnp.sum(jnp.where(iota_k == 1, rank, 0), axis=1, keepdims=True)
        pg1 = jnp.sum(jnp.where(i1 == iota_e, pstart, 0.0), axis=1,
                      keepdims=True)
        pg2 = jnp.sum(jnp.where(i2 == iota_e, pstart, 0.0), axis=1,
                      keepdims=True)
        pos0 = pg1.astype(jnp.int32) + r1
        pos1 = pg2.astype(jnp.int32) + r2
        pos_ref[...] = jnp.concatenate([pos0, pos1], axis=1)

        cb = pblk + nbef
        bf = lax.broadcasted_iota(jnp.int32, (NB, E), 0).astype(jnp.float32)
        blk = jnp.sum((cb <= bf).astype(jnp.int32), axis=1, keepdims=True)
        blk_ref[...] = jnp.minimum(blk, E - 1)
        nb_ref[...] = jnp.sum(nbe, axis=1, keepdims=True)


def _router_finalize(x, Wr, bias2):
    last = NTB - 1
    return pl.pallas_call(
        _router_body,
        grid=(NTB + 1,),
        in_specs=[
            pl.BlockSpec((BTR, D), lambda t: (jnp.minimum(t, last), 0)),
            pl.BlockSpec((E, D), lambda t: (0, 0)),
            pl.BlockSpec((1, E), lambda t: (0, 0)),
        ],
        out_specs=[
            pl.BlockSpec((BTR, K), lambda t: (jnp.minimum(t, last), 0)),
            pl.BlockSpec((T, K), lambda t: (0, 0)),
            pl.BlockSpec((NB, 1), lambda t: (0, 0)),
            pl.BlockSpec((1, 1), lambda t: (0, 0)),
        ],
        out_shape=[
            jax.ShapeDtypeStruct((T, K), jnp.float32),
            jax.ShapeDtypeStruct((T, K), jnp.int32),
            jax.ShapeDtypeStruct((NB, 1), jnp.int32),
            jax.ShapeDtypeStruct((1, 1), jnp.int32),
        ],
        scratch_shapes=[
            pltpu.VMEM((1, E), jnp.float32),
            pltpu.VMEM((T, K), jnp.int32),
            pltpu.VMEM((T, K), jnp.int32),
        ],
    )(x, Wr, bias2)


@functools.cache
def _make_sc_scatter_xs():
    per_w = T // NW
    mesh = plsc.VectorSubcoreMesh(core_axis_name="c", subcore_axis_name="s")

    @functools.partial(
        pl.kernel,
        mesh=mesh,
        out_type=jax.ShapeDtypeStruct((NP, D), jnp.float32),
        scratch_types=[
            pltpu.VMEM((K * per_w,), jnp.int32),
            pltpu.VMEM((per_w,), jnp.int32),
            pltpu.VMEM((per_w,), jnp.int32),
            pltpu.VMEM((per_w, D), jnp.float32),
            pltpu.SemaphoreType.DMA,
        ],
        compiler_params=pltpu.CompilerParams(needs_layout_passes=False),
    )
    def _scatter(pos_hbm, x_hbm, xs_hbm, idxall_v, idx0_v, idx1_v, rows_v,
                 sem):
        wid = lax.axis_index("s") * NC + lax.axis_index("c")
        base = wid * per_w
        pltpu.sync_copy(pos_hbm.at[pl.ds(base * K, per_w * K)], idxall_v)
        pltpu.sync_copy(x_hbm.at[pl.ds(base, per_w)], rows_v)
        iota = lax.broadcasted_iota(jnp.int32, (L,), 0)
        for c in range(per_w // L):
            lanes = iota * K + c * (L * K)
            idx0_v[pl.ds(c * L, L)] = plsc.load_gather(idxall_v, [lanes])
            idx1_v[pl.ds(c * L, L)] = plsc.load_gather(idxall_v, [lanes + 1])
        c0 = pltpu.async_copy(rows_v, xs_hbm.at[idx0_v], sem)
        c1 = pltpu.async_copy(rows_v, xs_hbm.at[idx1_v], sem)
        c0.wait()
        c1.wait()

    return _scatter


@functools.cache
def _make_sc_row_gather(nrows, ncols, chunk):
    per_w = nrows // NW
    nch = per_w // chunk
    mesh = plsc.VectorSubcoreMesh(core_axis_name="c", subcore_axis_name="s")

    @functools.partial(
        pl.kernel,
        mesh=mesh,
        out_type=jax.ShapeDtypeStruct((nrows, ncols), jnp.float32),
        scratch_types=[
            pltpu.VMEM((per_w,), jnp.int32),
            pltpu.VMEM((chunk, ncols), jnp.float32),
            pltpu.VMEM((chunk, ncols), jnp.float32),
            pltpu.SemaphoreType.DMA,
            pltpu.SemaphoreType.DMA,
        ],
        compiler_params=pltpu.CompilerParams(needs_layout_passes=False),
    )
    def _gather(idx_hbm, table_hbm, out_hbm, idx_v, buf0, buf1, sem_g, sem_w):
        wid = lax.axis_index("s") * NC + lax.axis_index("c")
        base = wid * per_w
        pltpu.sync_copy(idx_hbm.at[pl.ds(base, per_w)], idx_v)
        bufs = [buf0, buf1]
        g = [None] * nch
        w = [None] * nch
        g[0] = pltpu.async_copy(
            table_hbm.at[idx_v.at[pl.ds(0, chunk)]], bufs[0], sem_g)
        for c in range(nch):
            g[c].wait()
            if c + 1 < nch:
                if c >= 1:
                    w[c - 1].wait()
                g[c + 1] = pltpu.async_copy(
                    table_hbm.at[idx_v.at[pl.ds((c + 1) * chunk, chunk)]],
                    bufs[(c + 1) % 2], sem_g)
            w[c] = pltpu.async_copy(
                bufs[c % 2], out_hbm.at[pl.ds(base + c * chunk, chunk)], sem_w)
        if nch >= 2:
            w[nch - 2].wait()
        w[nch - 1].wait()

    return _gather


def _ffn_body(blk_ref, nb_ref, xs_ref, wg_ref, wu_ref, wd_ref, yp_ref):
    b = pl.program_id(0)

    @pl.when(b < nb_ref[0])
    def _():
        xb = xs_ref[...]
        hg = lax.dot_general(xb, wg_ref[0], (((1,), (1,)), ((), ())),
                             preferred_element_type=jnp.float32)
        hu = lax.dot_general(xb, wu_ref[0], (((1,), (1,)), ((), ())),
                             preferred_element_type=jnp.float32)
        h = hg * lax.logistic(hg) * hu
        yp_ref[...] = lax.dot_general(h, wd_ref[0], (((1,), (1,)), ((), ())),
                                      preferred_element_type=jnp.float32)


def _grouped_ffn(blk, nb, xs, Wg, Wu, Wd):
    grid_spec = pltpu.PrefetchScalarGridSpec(
        num_scalar_prefetch=2,
        grid=(NB,),
        in_specs=[
            pl.BlockSpec((BT, D), lambda b, blk, nb: (b, 0)),
            pl.BlockSpec((1, FF, D), lambda b, blk, nb: (blk[b], 0, 0)),
            pl.BlockSpec((1, FF, D), lambda b, blk, nb: (blk[b], 0, 0)),
            pl.BlockSpec((1, D, FF), lambda b, blk, nb: (blk[b], 0, 0)),
        ],
        out_specs=pl.BlockSpec((BT, D), lambda b, blk, nb: (b, 0)),
    )
    return pl.pallas_call(
        _ffn_body,
        grid_spec=grid_spec,
        out_shape=jax.ShapeDtypeStruct((NP, D), jnp.float32),
    )(blk, nb, xs, Wg, Wu, Wd)


def _combine_body(ypt_ref, scores_ref, y_ref):
    s = scores_ref[...]
    iota_k = lax.broadcasted_iota(jnp.int32, (BTC, K), 1)
    s0 = jnp.sum(jnp.where(iota_k == 0, s, 0.0), axis=1, keepdims=True)
    s1 = jnp.sum(jnp.where(iota_k == 1, s, 0.0), axis=1, keepdims=True)
    ri = lax.broadcasted_iota(jnp.int32, (BTC, K * BTC), 0)
    ci = lax.broadcasted_iota(jnp.int32, (BTC, K * BTC), 1)
    a = (jnp.where(ci == 2 * ri, s0, 0.0)
         + jnp.where(ci == 2 * ri + 1, s1, 0.0))
    y_ref[...] = lax.dot_general(a, ypt_ref[...], (((1,), (0,)), ((), ())),
                                 preferred_element_type=jnp.float32)


def _combine(ypt, scores):
    return pl.pallas_call(
        _combine_body,
        grid=(T // BTC,),
        in_specs=[
            pl.BlockSpec((K * BTC, D), lambda t: (t, 0)),
            pl.BlockSpec((BTC, K), lambda t: (t, 0)),
        ],
        out_specs=pl.BlockSpec((BTC, D), lambda t: (t, 0)),
        out_shape=jax.ShapeDtypeStruct((T, D), jnp.float32),
    )(ypt, scores)


def kernel(x, Wr, Wg, Wu, Wd, expert_bias):
    bias2 = expert_bias.reshape(1, E)
    scores, pos, blk, nb = _router_finalize(x, Wr, bias2)
    posflat = pos.reshape(TK)
    xs = _make_sc_scatter_xs()(posflat, x)
    yp = _grouped_ffn(blk.reshape(NB), nb.reshape(1), xs, Wg, Wu, Wd)
    ypt = _make_sc_row_gather(TK, D, 32)(posflat, yp)
    y = _combine(ypt, scores)
    return y
